# Initial kernel scaffold; baseline (speedup 1.0000x reference)
import jax
import jax.numpy as jnp
from jax.experimental import pallas as pl


def kernel(x_nchw, conv_wk0, conv_aff0, conv_wk1, conv_aff1, conv_wk2, conv_aff2, conv_wk3, conv_aff3, conv_wk4, conv_aff4, conv_wk5, conv_aff5, fw1, fb1, fw2, fb2):
    raise NotImplementedError("write your pallas kernel here")


# speedup vs baseline: 1.0668x; 1.0668x over previous
import jax
import jax.numpy as jnp
from jax.experimental import pallas as pl
from jax.experimental.pallas import tpu as pltpu


_CFG = (
    (32, 32,   3,  32, False),
    (32, 32,  32,  32, True),
    (16, 16,  32,  64, False),
    (16, 16,  64,  64, True),
    ( 8,  8,  64, 128, False),
    ( 8,  8, 128, 128, True),
)
_FEAT_DIM = 128 * 4 * 4


def _pick_batch_block(batch):
    for bb in (8, 4, 2):
        if batch % bb == 0 and batch >= 2 * bb:
            return bb
    return 1


def _zero_halo(xk):
    b, hp, w, kc = xk.shape
    h = hp - 4
    z = jnp.zeros((b, 2, w, kc), jnp.bfloat16)
    xk[:, 0:2, :, :] = z
    xk[:, hp - 2:hp, :, :] = z
    zc = jnp.zeros((b, h, 2, kc), jnp.bfloat16)
    xk[:, 2:2 + h, 0:2, :] = zc
    xk[:, 2:2 + h, w - 2:w, :] = zc


def _fill_dx_stacked(xk, src, cin):
    b, rows, w, _ = src.shape
    for dx in range(5):
        t_lo = max(0, 2 - dx)
        t_hi = min(w, w + 2 - dx)
        s_lo = t_lo + dx - 2
        s_hi = t_hi + dx - 2
        xk[:, 2:2 + rows, t_lo:t_hi, dx * cin:(dx + 1) * cin] = src[:, :, s_lo:s_hi, :]


def _make_body(b_blk):
    def body(x_ref,
             w1, a1, w2, a2, w3, a3, w4, a4, w5, a5, w6, a6,
             fw1, fb1, fw2, fb2,
             o_ref,
             xk1, xk2, xk3, xk4, xk5, xk6,
             p2, p4, p6, fc_in):
        wrefs = (w1, w2, w3, w4, w5, w6)
        arefs = (a1, a2, a3, a4, a5, a6)
        xks = (xk1, xk2, xk3, xk4, xk5, xk6)
        pools = {1: p2, 3: p4, 5: p6}

        for xk in xks:
            _zero_halo(xk)
        _fill_dx_stacked(xks[0], x_ref[...], 3)

        y = None
        for li, (H, W, Cin, Cout, pool) in enumerate(_CFG):
            kc = 5 * Cin
            hp = H + 4
            lhs = xks[li][...].reshape(b_blk * hp * W, kc)
            yp = jnp.dot(lhs, wrefs[li][...],
                         preferred_element_type=jnp.float32)
            yp = yp.reshape(b_blk, hp, W, 5 * Cout)
            acc = yp[:, 0:H, :, 0:Cout]
            for dy in range(1, 5):
                acc = acc + yp[:, dy:dy + H, :, dy * Cout:(dy + 1) * Cout]
            aff = arefs[li][...]
            y = (jnp.maximum(acc + aff[0].reshape(1, 1, 1, Cout), 0.0)
                 * aff[1].reshape(1, 1, 1, Cout) + aff[2].reshape(1, 1, 1, Cout))
            if pool:
                pref = pools[li]
                pref[...] = y
                even = pref[:, :, pl.ds(0, W // 2, 2), :]
                odd = pref[:, :, pl.ds(1, W // 2, 2), :]
                cm = jnp.maximum(even, odd)
                cm = cm.reshape(b_blk, H // 2, 2, W // 2, Cout)
                y = jnp.maximum(cm[:, :, 0], cm[:, :, 1])
            yb = y.astype(jnp.bfloat16)
            if li + 1 < 6:
                _fill_dx_stacked(xks[li + 1], yb, Cout)
            else:
                for i in range(4):
                    for j in range(4):
                        s = i * 4 + j
                        fc_in[:, s * 128:(s + 1) * 128] = yb[:, i, j, :]

        f = fc_in[...]
        z = jnp.dot(f, fw1[...], preferred_element_type=jnp.float32) + fb1[...]
        z = jnp.maximum(z, 0.0)
        out = jnp.dot(z, fw2[...], preferred_element_type=jnp.float32) + fb2[...]
        o_ref[...] = out

    return body


def kernel(x_nchw, conv_wk0, conv_aff0, conv_wk1, conv_aff1, conv_wk2,
           conv_aff2, conv_wk3, conv_aff3, conv_wk4, conv_aff4, conv_wk5,
           conv_aff5, fw1, fb1, fw2, fb2):
    B = x_nchw.shape[0]
    x = jnp.transpose(x_nchw.astype(jnp.bfloat16), (0, 2, 3, 1))
    wks = (conv_wk0, conv_wk1, conv_wk2, conv_wk3, conv_wk4, conv_wk5)
    affs = (conv_aff0, conv_aff1, conv_aff2, conv_aff3, conv_aff4, conv_aff5)
    wns = [jnp.transpose(wk, (1, 0, 2)).reshape(wk.shape[1], 5 * wk.shape[2])
           for wk in wks]

    b_blk = _pick_batch_block(B)
    grid = (B // b_blk,)
    H1 = fw1.shape[1]
    NC = fw2.shape[1]

    in_specs = [pl.BlockSpec((b_blk, 32, 32, 3), lambda i: (i, 0, 0, 0))]
    args = [x]
    for wn, aff in zip(wns, affs):
        in_specs.append(pl.BlockSpec(wn.shape, lambda i: (0, 0)))
        in_specs.append(pl.BlockSpec(aff.shape, lambda i: (0, 0)))
        args.extend([wn, aff])
    in_specs.extend([
        pl.BlockSpec(fw1.shape, lambda i: (0, 0)),
        pl.BlockSpec((1, H1), lambda i: (0, 0)),
        pl.BlockSpec(fw2.shape, lambda i: (0, 0)),
        pl.BlockSpec((1, NC), lambda i: (0, 0)),
    ])
    args.extend([fw1, fb1.reshape(1, H1), fw2, fb2.reshape(1, NC)])

    scratch_shapes = [
        pltpu.VMEM((b_blk, H + 4, W, 5 * Cin), jnp.bfloat16)
        for (H, W, Cin, Cout, pool) in _CFG
    ]
    scratch_shapes += [
        pltpu.VMEM((b_blk, H, W, Cout), jnp.float32)
        for (H, W, Cin, Cout, pool) in _CFG if pool
    ]
    scratch_shapes.append(pltpu.VMEM((b_blk, _FEAT_DIM), jnp.bfloat16))

    out = pl.pallas_call(
        _make_body(b_blk),
        out_shape=jax.ShapeDtypeStruct((B, NC), jnp.float32),
        grid=grid,
        in_specs=in_specs,
        out_specs=pl.BlockSpec((b_blk, NC), lambda i: (i, 0)),
        scratch_shapes=scratch_shapes,
        compiler_params=pltpu.CompilerParams(
            dimension_semantics=("parallel",),
            vmem_limit_bytes=48 * 1024 * 1024,
        ),
    )(*args)
    return out


# speedup vs baseline: 1.1906x; 1.1161x over previous
import jax
import jax.numpy as jnp
from jax.experimental import pallas as pl
from jax.experimental.pallas import tpu as pltpu


_CFG = (
    (32, 32,   3,  32, False),
    (32, 32,  32,  32, True),
    (16, 16,  32,  64, False),
    (16, 16,  64,  64, True),
    ( 8,  8,  64, 128, False),
    ( 8,  8, 128, 128, True),
)
_FEAT_DIM = 128 * 4 * 4


def _pick_batch_block(batch):
    for bb in (8, 4, 2):
        if batch % bb == 0 and batch >= 2 * bb:
            return bb
    return 1


def _zero_halo(xk):
    b, hp, w, kc = xk.shape
    h = hp - 4
    z = jnp.zeros((b, 2, w, kc), jnp.bfloat16)
    xk[:, 0:2, :, :] = z
    xk[:, hp - 2:hp, :, :] = z
    zc = jnp.zeros((b, h, 2, kc), jnp.bfloat16)
    xk[:, 2:2 + h, 0:2, :] = zc
    xk[:, 2:2 + h, w - 2:w, :] = zc


def _fill_dx_stacked(xk, src, cin):
    b, rows, w, _ = src.shape
    for dx in range(5):
        t_lo = max(0, 2 - dx)
        t_hi = min(w, w + 2 - dx)
        s_lo = t_lo + dx - 2
        s_hi = t_hi + dx - 2
        xk[:, 2:2 + rows, t_lo:t_hi, dx * cin:(dx + 1) * cin] = src[:, :, s_lo:s_hi, :]


def _make_body(b_blk):
    def body(x_ref,
             w1, a1, w2, a2, w3, a3, w4, a4, w5, a5, w6, a6,
             fw1, fb1, fw2, fb2,
             o_ref,
             xk1, xk2, xk3, xk4, xk5, xk6,
             p2, p4, p6, fc_in):
        wrefs = (w1, w2, w3, w4, w5, w6)
        arefs = (a1, a2, a3, a4, a5, a6)
        xks = (xk1, xk2, xk3, xk4, xk5, xk6)
        pools = {1: p2, 3: p4, 5: p6}

        for xk in xks:
            _zero_halo(xk)
        b_blk = xks[0].shape[0]
        xt = jnp.transpose(x_ref[...].reshape(b_blk, 3, 32, 32), (0, 2, 3, 1))
        _fill_dx_stacked(xks[0], xt, 3)

        y = None
        for li, (H, W, Cin, Cout, pool) in enumerate(_CFG):
            kc = 5 * Cin
            hp = H + 4
            lhs = xks[li][...].reshape(b_blk * hp * W, kc)
            yp = jnp.dot(lhs, wrefs[li][...],
                         preferred_element_type=jnp.float32)
            yp = yp.reshape(b_blk, hp, W, 5 * Cout)
            acc = yp[:, 0:H, :, 0:Cout]
            for dy in range(1, 5):
                acc = acc + yp[:, dy:dy + H, :, dy * Cout:(dy + 1) * Cout]
            aff = arefs[li][...]
            y = (jnp.maximum(acc + aff[0].reshape(1, 1, 1, Cout), 0.0)
                 * aff[1].reshape(1, 1, 1, Cout) + aff[2].reshape(1, 1, 1, Cout))
            if pool:
                pref = pools[li]
                pref[...] = y
                even = pref[:, :, pl.ds(0, W // 2, 2), :]
                odd = pref[:, :, pl.ds(1, W // 2, 2), :]
                cm = jnp.maximum(even, odd)
                cm = cm.reshape(b_blk, H // 2, 2, W // 2, Cout)
                y = jnp.maximum(cm[:, :, 0], cm[:, :, 1])
            yb = y.astype(jnp.bfloat16)
            if li + 1 < 6:
                _fill_dx_stacked(xks[li + 1], yb, Cout)
            else:
                for i in range(4):
                    for j in range(4):
                        s = i * 4 + j
                        fc_in[:, s * 128:(s + 1) * 128] = yb[:, i, j, :]

        f = fc_in[...]
        z = jnp.dot(f, fw1[...], preferred_element_type=jnp.float32) + fb1[...]
        z = jnp.maximum(z, 0.0)
        out = jnp.dot(z, fw2[...], preferred_element_type=jnp.float32) + fb2[...]
        o_ref[...] = out

    return body


def kernel(x_nchw, conv_wk0, conv_aff0, conv_wk1, conv_aff1, conv_wk2,
           conv_aff2, conv_wk3, conv_aff3, conv_wk4, conv_aff4, conv_wk5,
           conv_aff5, fw1, fb1, fw2, fb2):
    B = x_nchw.shape[0]
    x = x_nchw.astype(jnp.bfloat16).reshape(B * 3, 32, 32)
    wks = (conv_wk0, conv_wk1, conv_wk2, conv_wk3, conv_wk4, conv_wk5)
    affs = (conv_aff0, conv_aff1, conv_aff2, conv_aff3, conv_aff4, conv_aff5)
    wns = [jnp.transpose(wk, (1, 0, 2)).reshape(wk.shape[1], 5 * wk.shape[2])
           for wk in wks]

    b_blk = _pick_batch_block(B)
    nsteps = B // b_blk
    H1 = fw1.shape[1]
    NC = fw2.shape[1]

    grid = (nsteps,)
    semantics = ("arbitrary",)
    xmap = lambda i: (i, 0, 0)
    omap = lambda i: (i, 0)
    wmap2 = lambda i: (0, 0)

    in_specs = [pl.BlockSpec((b_blk * 3, 32, 32), xmap)]
    args = [x]
    for wn, aff in zip(wns, affs):
        in_specs.append(pl.BlockSpec(wn.shape, wmap2))
        in_specs.append(pl.BlockSpec(aff.shape, wmap2))
        args.extend([wn, aff])
    in_specs.extend([
        pl.BlockSpec(fw1.shape, wmap2),
        pl.BlockSpec((1, H1), wmap2),
        pl.BlockSpec(fw2.shape, wmap2),
        pl.BlockSpec((1, NC), wmap2),
    ])
    args.extend([fw1, fb1.reshape(1, H1), fw2, fb2.reshape(1, NC)])

    scratch_shapes = [
        pltpu.VMEM((b_blk, H + 4, W, 5 * Cin), jnp.bfloat16)
        for (H, W, Cin, Cout, pool) in _CFG
    ]
    scratch_shapes += [
        pltpu.VMEM((b_blk, H, W, Cout), jnp.float32)
        for (H, W, Cin, Cout, pool) in _CFG if pool
    ]
    scratch_shapes.append(pltpu.VMEM((b_blk, _FEAT_DIM), jnp.bfloat16))

    out = pl.pallas_call(
        _make_body(b_blk),
        out_shape=jax.ShapeDtypeStruct((B, NC), jnp.float32),
        grid=grid,
        in_specs=in_specs,
        out_specs=pl.BlockSpec((b_blk, NC), omap),
        scratch_shapes=scratch_shapes,
        compiler_params=pltpu.CompilerParams(
            dimension_semantics=semantics,
            vmem_limit_bytes=48 * 1024 * 1024,
        ),
    )(*args)
    return out


# speedup vs baseline: 1.7307x; 1.4537x over previous
import jax
import jax.numpy as jnp
from jax.experimental import pallas as pl
from jax.experimental.pallas import tpu as pltpu


_CFG = (
    (32, 32,   3,  32, False),
    (32, 32,  32,  32, True),
    (16, 16,  32,  64, False),
    (16, 16,  64,  64, True),
    ( 8,  8,  64, 128, False),
    ( 8,  8, 128, 128, True),
)
_FEAT_DIM = 128 * 4 * 4


def _pick_batch_block(batch):
    for bb in (8, 4, 2):
        if batch % bb == 0 and batch >= 2 * bb:
            return bb
    return 1


def _dx_stack_value(y):
    b, rows, w, c = y.shape
    parts = []
    for dx in range(5):
        if dx < 2:
            k = 2 - dx
            z = jnp.zeros((b, rows, k, c), jnp.bfloat16)
            parts.append(jnp.concatenate([z, y[:, :, :w - k, :]], axis=2))
        elif dx == 2:
            parts.append(y)
        else:
            k = dx - 2
            z = jnp.zeros((b, rows, k, c), jnp.bfloat16)
            parts.append(jnp.concatenate([y[:, :, k:, :], z], axis=2))
    return jnp.concatenate(parts, axis=3)


def _pad_rows(y):
    b, rows, w, c = y.shape
    z = jnp.zeros((b, 2, w, c), jnp.bfloat16)
    return jnp.concatenate([z, y, z], axis=1)


def _make_body(b_blk):
    def body(x_ref,
             w1, a1, w2, a2, w3, a3, w4, a4, w5, a5, w6, a6,
             fw1, fb1, fw2, fb2,
             o_ref,
             p2, p4, p6, fc_in):
        wrefs = (w1, w2, w3, w4, w5, w6)
        arefs = (a1, a2, a3, a4, a5, a6)
        pools = {1: p2, 3: p4, 5: p6}

        xt = jnp.transpose(x_ref[...].reshape(b_blk, 3, 32, 32), (0, 2, 3, 1))
        yb = xt

        for li, (H, W, Cin, Cout, pool) in enumerate(_CFG):
            kc = 5 * Cin
            hp = H + 4
            lhs = _dx_stack_value(_pad_rows(yb)).reshape(b_blk * hp * W, kc)
            yp = jnp.dot(lhs, wrefs[li][...],
                         preferred_element_type=jnp.float32)
            yp = yp.reshape(b_blk, hp, W, 5 * Cout)
            acc = yp[:, 0:H, :, 0:Cout]
            for dy in range(1, 5):
                acc = acc + yp[:, dy:dy + H, :, dy * Cout:(dy + 1) * Cout]
            aff = arefs[li][...]
            y = (jnp.maximum(acc + aff[0].reshape(1, 1, 1, Cout), 0.0)
                 * aff[1].reshape(1, 1, 1, Cout) + aff[2].reshape(1, 1, 1, Cout))
            if pool:
                rm = y.reshape(b_blk, H // 2, 2, W, Cout)
                rm = jnp.maximum(rm[:, :, 0], rm[:, :, 1])
                pref = pools[li]
                pref[...] = rm
                even = pref[:, :, pl.ds(0, W // 2, 2), :]
                odd = pref[:, :, pl.ds(1, W // 2, 2), :]
                y = jnp.maximum(even, odd)
            yb = y.astype(jnp.bfloat16)
            if li + 1 == 6:
                for i in range(4):
                    for j in range(4):
                        s = i * 4 + j
                        fc_in[:, s * 128:(s + 1) * 128] = yb[:, i, j, :]

        f = fc_in[...]
        z = jnp.dot(f, fw1[...], preferred_element_type=jnp.float32) + fb1[...]
        z = jnp.maximum(z, 0.0)
        out = jnp.dot(z, fw2[...], preferred_element_type=jnp.float32) + fb2[...]
        o_ref[...] = out

    return body


def kernel(x_nchw, conv_wk0, conv_aff0, conv_wk1, conv_aff1, conv_wk2,
           conv_aff2, conv_wk3, conv_aff3, conv_wk4, conv_aff4, conv_wk5,
           conv_aff5, fw1, fb1, fw2, fb2):
    B = x_nchw.shape[0]
    x = x_nchw.astype(jnp.bfloat16).reshape(B * 3, 32, 32)
    wks = (conv_wk0, conv_wk1, conv_wk2, conv_wk3, conv_wk4, conv_wk5)
    affs = (conv_aff0, conv_aff1, conv_aff2, conv_aff3, conv_aff4, conv_aff5)
    wns = [jnp.transpose(wk, (1, 0, 2)).reshape(wk.shape[1], 5 * wk.shape[2])
           for wk in wks]

    b_blk = _pick_batch_block(B)
    nsteps = B // b_blk
    H1 = fw1.shape[1]
    NC = fw2.shape[1]

    grid = (nsteps,)
    semantics = ("arbitrary",)
    xmap = lambda i: (i, 0, 0)
    omap = lambda i: (i, 0)
    wmap2 = lambda i: (0, 0)

    in_specs = [pl.BlockSpec((b_blk * 3, 32, 32), xmap)]
    args = [x]
    for wn, aff in zip(wns, affs):
        in_specs.append(pl.BlockSpec(wn.shape, wmap2))
        in_specs.append(pl.BlockSpec(aff.shape, wmap2))
        args.extend([wn, aff])
    in_specs.extend([
        pl.BlockSpec(fw1.shape, wmap2),
        pl.BlockSpec((1, H1), wmap2),
        pl.BlockSpec(fw2.shape, wmap2),
        pl.BlockSpec((1, NC), wmap2),
    ])
    args.extend([fw1, fb1.reshape(1, H1), fw2, fb2.reshape(1, NC)])

    scratch_shapes = [
        pltpu.VMEM((b_blk, H // 2, W, Cout), jnp.float32)
        for (H, W, Cin, Cout, pool) in _CFG if pool
    ]
    scratch_shapes.append(pltpu.VMEM((b_blk, _FEAT_DIM), jnp.bfloat16))

    out = pl.pallas_call(
        _make_body(b_blk),
        out_shape=jax.ShapeDtypeStruct((B, NC), jnp.float32),
        grid=grid,
        in_specs=in_specs,
        out_specs=pl.BlockSpec((b_blk, NC), omap),
        scratch_shapes=scratch_shapes,
        compiler_params=pltpu.CompilerParams(
            dimension_semantics=semantics,
            vmem_limit_bytes=48 * 1024 * 1024,
        ),
    )(*args)
    return out


# speedup vs baseline: 2.4035x; 1.3887x over previous
import jax
import jax.numpy as jnp
from jax.experimental import pallas as pl
from jax.experimental.pallas import tpu as pltpu


_CFG = (
    (32, 32,   3,  32, False),
    (32, 32,  32,  32, True),
    (16, 16,  32,  64, False),
    (16, 16,  64,  64, True),
    ( 8,  8,  64, 128, False),
    ( 8,  8, 128, 128, True),
)
_FEAT_DIM = 128 * 4 * 4


def _pick_batch_block(batch):
    for bb in (8, 4, 2):
        if batch % bb == 0 and batch >= 2 * bb:
            return bb
    return 1


def _dx_stack_value(y):
    b, rows, w, c = y.shape
    parts = []
    for dx in range(5):
        if dx < 2:
            k = 2 - dx
            z = jnp.zeros((b, rows, k, c), jnp.bfloat16)
            parts.append(jnp.concatenate([z, y[:, :, :w - k, :]], axis=2))
        elif dx == 2:
            parts.append(y)
        else:
            k = dx - 2
            z = jnp.zeros((b, rows, k, c), jnp.bfloat16)
            parts.append(jnp.concatenate([y[:, :, k:, :], z], axis=2))
    kc = 5 * c
    kpad = (-kc) % 128
    if kpad:
        parts.append(jnp.zeros((b, rows, w, kpad), jnp.bfloat16))
    return jnp.concatenate(parts, axis=3)


def _pad_rows(y):
    b, rows, w, c = y.shape
    z = jnp.zeros((b, 2, w, c), jnp.bfloat16)
    return jnp.concatenate([z, y, z], axis=1)


def _conv_layer0(xr, w0_ref, a0_ref):
    b = xr.shape[0]
    zw = jnp.zeros((b, 3, 32, 2), jnp.bfloat16)
    xp = jnp.concatenate([zw, xr, zw], axis=3)
    lanes = jnp.concatenate([xp[:, 0], xp[:, 1], xp[:, 2]], axis=2)
    zr = jnp.zeros((b, 4, 108), jnp.bfloat16)
    lhs = jnp.concatenate([jnp.zeros((b, 2, 108), jnp.bfloat16), lanes,
                           jnp.zeros((b, 2, 108), jnp.bfloat16), zr], axis=1)
    yp = jnp.dot(lhs.reshape(b * 40, 108), w0_ref[...],
                 preferred_element_type=jnp.float32)
    yp = yp.reshape(b, 40, 5 * 1024)
    acc = yp[:, 0:32, 0:1024]
    for dy in range(1, 5):
        acc = acc + yp[:, dy:dy + 32, dy * 1024:(dy + 1) * 1024]
    aff = a0_ref[...]
    y = (jnp.maximum(acc + aff[0].reshape(1, 1, 1024), 0.0)
         * aff[1].reshape(1, 1, 1024) + aff[2].reshape(1, 1, 1024))
    return y.reshape(b, 32, 32, 32)


def _conv_stack(xr, wrefs, arefs, pools, fc_in, row0):
    b2 = xr.shape[0]
    yb = None
    for li, (H, W, Cin, Cout, pool) in enumerate(_CFG):
        if li == 0:
            y = _conv_layer0(xr, wrefs[0], arefs[0])
        else:
            kc = 5 * Cin + (-5 * Cin) % 128
            hp = H + 4
            lhs = _dx_stack_value(_pad_rows(yb)).reshape(b2 * hp * W, kc)
            yp = jnp.dot(lhs, wrefs[li][...],
                         preferred_element_type=jnp.float32)
            yp = yp.reshape(b2, hp, W, 5 * Cout)
            acc = yp[:, 0:H, :, 0:Cout]
            for dy in range(1, 5):
                acc = acc + yp[:, dy:dy + H, :, dy * Cout:(dy + 1) * Cout]
            aff = arefs[li][...]
            y = (jnp.maximum(acc + aff[0].reshape(1, 1, 1, Cout), 0.0)
                 * aff[1].reshape(1, 1, 1, Cout) + aff[2].reshape(1, 1, 1, Cout))
        if pool:
            rm = y.reshape(b2, H // 2, 2, W, Cout)
            rm = jnp.maximum(rm[:, :, 0], rm[:, :, 1])
            pref = pools[li]
            pref[...] = rm
            even = pref[:, :, pl.ds(0, W // 2, 2), :]
            odd = pref[:, :, pl.ds(1, W // 2, 2), :]
            y = jnp.maximum(even, odd)
        yb = y.astype(jnp.bfloat16)
        if li + 1 == 6:
            for i in range(4):
                for j in range(4):
                    s = i * 4 + j
                    fc_in[row0:row0 + b2, s * 128:(s + 1) * 128] = yb[:, i, j, :]


def _make_body(b_blk):
    def body(x_ref,
             w1, a1, w2, a2, w3, a3, w4, a4, w5, a5, w6, a6,
             fw1, fb1, fw2, fb2,
             o_ref,
             p2, p4, p6, fc_in):
        wrefs = (w1, w2, w3, w4, w5, w6)
        arefs = (a1, a2, a3, a4, a5, a6)

        xr = x_ref[...].reshape(b_blk, 3, 32, 32)
        _conv_stack(xr, wrefs, arefs, {1: p2, 3: p4, 5: p6}, fc_in, 0)

        f = fc_in[...]
        z = jnp.dot(f, fw1[...], preferred_element_type=jnp.float32) + fb1[...]
        z = jnp.maximum(z, 0.0)
        out = jnp.dot(z, fw2[...], preferred_element_type=jnp.float32) + fb2[...]
        o_ref[...] = out

    return body


def kernel(x_nchw, conv_wk0, conv_aff0, conv_wk1, conv_aff1, conv_wk2,
           conv_aff2, conv_wk3, conv_aff3, conv_wk4, conv_aff4, conv_wk5,
           conv_aff5, fw1, fb1, fw2, fb2):
    B = x_nchw.shape[0]
    x = x_nchw.astype(jnp.bfloat16).reshape(B * 3, 32, 32)
    wks = (conv_wk0, conv_wk1, conv_wk2, conv_wk3, conv_wk4, conv_wk5)
    affs = (conv_aff0, conv_aff1, conv_aff2, conv_aff3, conv_aff4, conv_aff5)
    wns = [jnp.transpose(wk, (1, 0, 2)).reshape(wk.shape[1], 5 * wk.shape[2])
           for wk in wks]
    wns = [jnp.pad(wn, ((0, (-wn.shape[0]) % 128), (0, 0))) for wn in wns]
    wk0r = conv_wk0.reshape(5, 5, 3, 32).astype(jnp.float32)
    tsel = (jnp.arange(36)[:, None, None]
            == jnp.arange(32)[None, :, None]
            + jnp.arange(5)[None, None, :]).astype(jnp.float32)
    wns[0] = jnp.einsum('pwx,yxco->cpywo', tsel, wk0r
                        ).reshape(108, 5120).astype(jnp.bfloat16)
    affs = (jnp.tile(conv_aff0, (1, 32)),) + affs[1:]

    b_blk = _pick_batch_block(B)
    nsteps = B // b_blk
    H1 = fw1.shape[1]
    NC = fw2.shape[1]

    grid = (nsteps,)
    semantics = ("arbitrary",)
    xmap = lambda i: (i, 0, 0)
    omap = lambda i: (i, 0)
    wmap2 = lambda i: (0, 0)

    in_specs = [pl.BlockSpec((b_blk * 3, 32, 32), xmap)]
    args = [x]
    for wn, aff in zip(wns, affs):
        in_specs.append(pl.BlockSpec(wn.shape, wmap2))
        in_specs.append(pl.BlockSpec(aff.shape, wmap2))
        args.extend([wn, aff])
    in_specs.extend([
        pl.BlockSpec(fw1.shape, wmap2),
        pl.BlockSpec((1, H1), wmap2),
        pl.BlockSpec(fw2.shape, wmap2),
        pl.BlockSpec((1, NC), wmap2),
    ])
    args.extend([fw1, fb1.reshape(1, H1), fw2, fb2.reshape(1, NC)])

    scratch_shapes = [
        pltpu.VMEM((b_blk, H // 2, W, Cout), jnp.float32)
        for (H, W, Cin, Cout, pool) in _CFG if pool
    ]
    scratch_shapes.append(pltpu.VMEM((b_blk, _FEAT_DIM), jnp.bfloat16))

    out = pl.pallas_call(
        _make_body(b_blk),
        out_shape=jax.ShapeDtypeStruct((B, NC), jnp.float32),
        grid=grid,
        in_specs=in_specs,
        out_specs=pl.BlockSpec((b_blk, NC), omap),
        scratch_shapes=scratch_shapes,
        compiler_params=pltpu.CompilerParams(
            dimension_semantics=semantics,
            vmem_limit_bytes=48 * 1024 * 1024,
        ),
    )(*args)
    return out


# speedup vs baseline: 2.6051x; 1.0839x over previous
import jax
import jax.numpy as jnp
from jax.experimental import pallas as pl
from jax.experimental.pallas import tpu as pltpu


_CFG = (
    (32, 32,   3,  32, False),
    (32, 32,  32,  32, True),
    (16, 16,  32,  64, False),
    (16, 16,  64,  64, True),
    ( 8,  8,  64, 128, False),
    ( 8,  8, 128, 128, True),
)
_FEAT_DIM = 128 * 4 * 4


def _pick_batch_block(batch):
    for bb in (8, 4, 2):
        if batch % bb == 0 and batch >= 2 * bb:
            return bb
    return 1


def _dx_stack_value(y):
    b, rows, w, c = y.shape
    parts = []
    for dx in range(5):
        if dx < 2:
            k = 2 - dx
            z = jnp.zeros((b, rows, k, c), jnp.bfloat16)
            parts.append(jnp.concatenate([z, y[:, :, :w - k, :]], axis=2))
        elif dx == 2:
            parts.append(y)
        else:
            k = dx - 2
            z = jnp.zeros((b, rows, k, c), jnp.bfloat16)
            parts.append(jnp.concatenate([y[:, :, k:, :], z], axis=2))
    kc = 5 * c
    kpad = (-kc) % 128
    if kpad:
        parts.append(jnp.zeros((b, rows, w, kpad), jnp.bfloat16))
    return jnp.concatenate(parts, axis=3)


def _pad_rows(y):
    b, rows, w, c = y.shape
    z = jnp.zeros((b, 2, w, c), jnp.bfloat16)
    return jnp.concatenate([z, y, z], axis=1)


def _conv_layer0(xr, w0_ref, a0_ref):
    b = xr.shape[0]
    zw = jnp.zeros((b, 3, 32, 2), jnp.bfloat16)
    xp = jnp.concatenate([zw, xr, zw], axis=3)
    lanes = jnp.concatenate([xp[:, 0], xp[:, 1], xp[:, 2]], axis=2)
    zr = jnp.zeros((b, 4, 108), jnp.bfloat16)
    lhs = jnp.concatenate([jnp.zeros((b, 2, 108), jnp.bfloat16), lanes,
                           jnp.zeros((b, 2, 108), jnp.bfloat16), zr], axis=1)
    yp = jnp.dot(lhs.reshape(b * 40, 108), w0_ref[...],
                 preferred_element_type=jnp.float32)
    yp = yp.reshape(b, 40, 5 * 1024)
    acc = yp[:, 0:32, 0:1024]
    for dy in range(1, 5):
        acc = acc + yp[:, dy:dy + 32, dy * 1024:(dy + 1) * 1024]
    aff = a0_ref[...]
    y = (jnp.maximum(acc + aff[0].reshape(1, 1, 1024), 0.0)
         * aff[1].reshape(1, 1, 1024) + aff[2].reshape(1, 1, 1024))
    return y.reshape(b, 32, 32, 32)


def _conv_stack(xr, wrefs, arefs, pools, fc_in, row0):
    b2 = xr.shape[0]
    yb = None
    for li, (H, W, Cin, Cout, pool) in enumerate(_CFG):
        if li == 0:
            y = _conv_layer0(xr, wrefs[0], arefs[0])
        else:
            kc = 5 * Cin + (-5 * Cin) % 128
            lhs = _dx_stack_value(yb).reshape(b2 * H * W, kc)
            yp = jnp.dot(lhs, wrefs[li][...],
                         preferred_element_type=jnp.float32)
            yp = yp.reshape(b2, H, W, 5 * Cout)
            acc = None
            for dy in range(5):
                lo = max(0, 2 - dy)
                hi = min(H, H + 2 - dy)
                part = yp[:, lo + dy - 2:hi + dy - 2, :,
                          dy * Cout:(dy + 1) * Cout]
                pads = []
                if lo:
                    pads.append(jnp.zeros((b2, lo, W, Cout), jnp.float32))
                pads.append(part)
                if H - hi:
                    pads.append(jnp.zeros((b2, H - hi, W, Cout), jnp.float32))
                part = jnp.concatenate(pads, axis=1) if len(pads) > 1 else part
                acc = part if acc is None else acc + part
            aff = arefs[li][...]
            y = (jnp.maximum(acc + aff[0].reshape(1, 1, 1, Cout), 0.0)
                 * aff[1].reshape(1, 1, 1, Cout) + aff[2].reshape(1, 1, 1, Cout))
        if pool:
            rm = y.reshape(b2, H // 2, 2, W, Cout)
            rm = jnp.maximum(rm[:, :, 0], rm[:, :, 1])
            pref = pools[li]
            pref[...] = rm
            even = pref[:, :, pl.ds(0, W // 2, 2), :]
            odd = pref[:, :, pl.ds(1, W // 2, 2), :]
            y = jnp.maximum(even, odd)
        yb = y.astype(jnp.bfloat16)
        if li + 1 == 6:
            for i in range(4):
                for j in range(4):
                    s = i * 4 + j
                    fc_in[row0:row0 + b2, s * 128:(s + 1) * 128] = yb[:, i, j, :]


def _make_body(b_blk):
    def body(x_ref,
             w1, a1, w2, a2, w3, a3, w4, a4, w5, a5, w6, a6,
             fw1, fb1, fw2, fb2,
             o_ref,
             p2, p4, p6, fc_in):
        wrefs = (w1, w2, w3, w4, w5, w6)
        arefs = (a1, a2, a3, a4, a5, a6)

        xr = x_ref[...].reshape(b_blk, 3, 32, 32)
        _conv_stack(xr, wrefs, arefs, {1: p2, 3: p4, 5: p6}, fc_in, 0)

        f = fc_in[...]
        z = jnp.dot(f, fw1[...], preferred_element_type=jnp.float32) + fb1[...]
        z = jnp.maximum(z, 0.0)
        out = jnp.dot(z, fw2[...], preferred_element_type=jnp.float32) + fb2[...]
        o_ref[...] = out

    return body


def kernel(x_nchw, conv_wk0, conv_aff0, conv_wk1, conv_aff1, conv_wk2,
           conv_aff2, conv_wk3, conv_aff3, conv_wk4, conv_aff4, conv_wk5,
           conv_aff5, fw1, fb1, fw2, fb2):
    B = x_nchw.shape[0]
    x = x_nchw.astype(jnp.bfloat16).reshape(B * 3, 32, 32)
    wks = (conv_wk0, conv_wk1, conv_wk2, conv_wk3, conv_wk4, conv_wk5)
    affs = (conv_aff0, conv_aff1, conv_aff2, conv_aff3, conv_aff4, conv_aff5)
    wns = [jnp.transpose(wk, (1, 0, 2)).reshape(wk.shape[1], 5 * wk.shape[2])
           for wk in wks]
    wns = [jnp.pad(wn, ((0, (-wn.shape[0]) % 128), (0, 0))) for wn in wns]
    wk0r = conv_wk0.reshape(5, 5, 3, 32).astype(jnp.float32)
    tsel = (jnp.arange(36)[:, None, None]
            == jnp.arange(32)[None, :, None]
            + jnp.arange(5)[None, None, :]).astype(jnp.float32)
    wns[0] = jnp.einsum('pwx,yxco->cpywo', tsel, wk0r
                        ).reshape(108, 5120).astype(jnp.bfloat16)
    affs = (jnp.tile(conv_aff0, (1, 32)),) + affs[1:]

    b_blk = _pick_batch_block(B)
    nsteps = B // b_blk
    H1 = fw1.shape[1]
    NC = fw2.shape[1]

    grid = (nsteps,)
    semantics = ("arbitrary",)
    xmap = lambda i: (i, 0, 0)
    omap = lambda i: (i, 0)
    wmap2 = lambda i: (0, 0)

    in_specs = [pl.BlockSpec((b_blk * 3, 32, 32), xmap)]
    args = [x]
    for wn, aff in zip(wns, affs):
        in_specs.append(pl.BlockSpec(wn.shape, wmap2))
        in_specs.append(pl.BlockSpec(aff.shape, wmap2))
        args.extend([wn, aff])
    in_specs.extend([
        pl.BlockSpec(fw1.shape, wmap2),
        pl.BlockSpec((1, H1), wmap2),
        pl.BlockSpec(fw2.shape, wmap2),
        pl.BlockSpec((1, NC), wmap2),
    ])
    args.extend([fw1, fb1.reshape(1, H1), fw2, fb2.reshape(1, NC)])

    scratch_shapes = [
        pltpu.VMEM((b_blk, H // 2, W, Cout), jnp.float32)
        for (H, W, Cin, Cout, pool) in _CFG if pool
    ]
    scratch_shapes.append(pltpu.VMEM((b_blk, _FEAT_DIM), jnp.bfloat16))

    out = pl.pallas_call(
        _make_body(b_blk),
        out_shape=jax.ShapeDtypeStruct((B, NC), jnp.float32),
        grid=grid,
        in_specs=in_specs,
        out_specs=pl.BlockSpec((b_blk, NC), omap),
        scratch_shapes=scratch_shapes,
        compiler_params=pltpu.CompilerParams(
            dimension_semantics=semantics,
            vmem_limit_bytes=48 * 1024 * 1024,
        ),
    )(*args)
    return out
